# baseline (speedup 1.0000x reference)
import functools

import jax
import jax.numpy as jnp
from jax import lax
from jax.experimental import pallas as pl
from jax.experimental.pallas import tpu as pltpu
from jax.experimental.pallas import tpu_sc as plsc

_N, _C, _P, _S = 4, 32, 256, 4096
_B = _N * _P
_NC, _NS = 1, 8
_NW = _NC * _NS
_BPW = _B // _NW
_BIG = 2**30
_SCHUNK = 128


def _dist_body(x_ref, cb_ref, idxf_ref, loss_ref):
    X = jnp.concatenate([x_ref[n] for n in range(_N)], axis=1)
    CB = cb_ref[:]
    csq = jnp.sum(CB * CB, axis=1, keepdims=True)
    A = jnp.concatenate([CB, csq], axis=1)
    Bm = jnp.concatenate(
        [-2.0 * X, jnp.ones((1, _B), jnp.float32)], axis=0)
    Ah = A.astype(jnp.bfloat16)
    Al = (A - Ah.astype(jnp.float32)).astype(jnp.bfloat16)
    Bh = Bm.astype(jnp.bfloat16)
    Bl = (Bm - Bh.astype(jnp.float32)).astype(jnp.bfloat16)
    Abig = jnp.concatenate([Ah, Ah, Al], axis=1)
    Bbig = jnp.concatenate([Bh, Bl, Bh], axis=0)
    nk = _S // _SCHUNK
    rows = lax.broadcasted_iota(jnp.int32, (_SCHUNK, _B), 0)
    m_run = None
    for k in range(nk):
        Ak = lax.slice(Abig, (k * _SCHUNK, 0), ((k + 1) * _SCHUNK, Abig.shape[1]))
        dk = lax.dot_general(
            Ak, Bbig, (((1,), (0,)), ((), ())),
            preferred_element_type=jnp.float32)
        mk = jnp.min(dk, axis=0)
        ik = jnp.min(jnp.where(dk <= mk[None, :], rows, _BIG), axis=0) + k * _SCHUNK
        if m_run is None:
            m_run, idx_run = mk, ik
        else:
            take = mk < m_run
            idx_run = jnp.where(take, ik, idx_run)
            m_run = jnp.where(take, mk, m_run)
    idxf_ref[:] = idx_run
    loss = (jnp.sum(m_run) + jnp.sum(x_ref[:] * x_ref[:])) * (1.0 / (_N * _C * _P))
    loss_ref[:, :] = loss.reshape(1, 1)


@functools.partial(
    pl.kernel,
    out_type=jax.ShapeDtypeStruct((_B, _C), jnp.float32),
    mesh=plsc.VectorSubcoreMesh(core_axis_name="c", subcore_axis_name="s", num_cores=1, num_subcores=8),
    scratch_types=[
        pltpu.VMEM((_BPW,), jnp.int32),
        pltpu.VMEM((_BPW, _C), jnp.float32),
        pltpu.SemaphoreType.DMA,
    ],
    compiler_params=pltpu.CompilerParams(use_tc_tiling_on_sc=False,
                                         skip_device_barrier=True),
)
def _sc_gather(cb_hbm, idx_hbm, out_hbm, idx_v, rows_v, sem):
    wid = lax.axis_index("s") * _NC + lax.axis_index("c")
    base = wid * _BPW
    pltpu.sync_copy(idx_hbm.at[pl.ds(base, _BPW)], idx_v)
    pltpu.async_copy(cb_hbm.at[idx_v], rows_v, sem).wait()
    pltpu.sync_copy(rows_v, out_hbm.at[pl.ds(base, _BPW)])


@jax.jit
def kernel(x, codebook):
    xr = x.reshape(_N, _C, _P)
    idx_flat, loss = pl.pallas_call(
        _dist_body,
        out_shape=[
            jax.ShapeDtypeStruct((_B,), jnp.int32),
            jax.ShapeDtypeStruct((1, 1), jnp.float32),
        ],
    )(xr, codebook)
    rows = _sc_gather(codebook, idx_flat)
    loss = loss.reshape(())
    indices = idx_flat.reshape(_N, 16, 16)
    output = rows.reshape(_N, _P, _C).transpose(0, 2, 1).reshape(x.shape)
    return (loss, loss, indices, output)


# speedup vs baseline: 1.0378x; 1.0378x over previous
import functools

import jax
import jax.numpy as jnp
from jax import lax
from jax.experimental import pallas as pl
from jax.experimental.pallas import tpu as pltpu
from jax.experimental.pallas import tpu_sc as plsc

_N, _C, _P, _S = 4, 32, 256, 4096
_B = _N * _P
_NC, _NS = 1, 16
_NW = _NC * _NS
_BPW = _B // _NW
_BIG = 2**30
_SCHUNK = 64


def _dist_body(x_ref, cb_ref, idxf_ref, loss_ref):
    X = jnp.concatenate([x_ref[n] for n in range(_N)], axis=1)
    CB = cb_ref[:]
    csq = jnp.sum(CB * CB, axis=1, keepdims=True)
    A = jnp.concatenate([CB, csq], axis=1)
    Bm = jnp.concatenate(
        [-2.0 * X, jnp.ones((1, _B), jnp.float32)], axis=0)
    Ah = A.astype(jnp.bfloat16)
    Al = (A - Ah.astype(jnp.float32)).astype(jnp.bfloat16)
    Bh = Bm.astype(jnp.bfloat16)
    Bl = (Bm - Bh.astype(jnp.float32)).astype(jnp.bfloat16)
    Abig = jnp.concatenate([Ah, Ah, Al], axis=1)
    Bbig = jnp.concatenate([Bh, Bl, Bh], axis=0)
    nk = _S // _SCHUNK
    rows = lax.broadcasted_iota(jnp.int32, (_SCHUNK, _B), 0)
    m_run = None
    for k in range(nk):
        Ak = lax.slice(Abig, (k * _SCHUNK, 0), ((k + 1) * _SCHUNK, Abig.shape[1]))
        dk = lax.dot_general(
            Ak, Bbig, (((1,), (0,)), ((), ())),
            preferred_element_type=jnp.float32)
        mk = jnp.min(dk, axis=0)
        ik = jnp.min(jnp.where(dk <= mk[None, :], rows, _BIG), axis=0) + k * _SCHUNK
        if m_run is None:
            m_run, idx_run = mk, ik
        else:
            take = mk < m_run
            idx_run = jnp.where(take, ik, idx_run)
            m_run = jnp.where(take, mk, m_run)
    idxf_ref[:] = idx_run
    loss = (jnp.sum(m_run) + jnp.sum(x_ref[:] * x_ref[:])) * (1.0 / (_N * _C * _P))
    loss_ref[:, :] = loss.reshape(1, 1)


@functools.partial(
    pl.kernel,
    out_type=jax.ShapeDtypeStruct((_B, _C), jnp.float32),
    mesh=plsc.VectorSubcoreMesh(core_axis_name="c", subcore_axis_name="s", num_cores=1),
    scratch_types=[
        pltpu.VMEM((_BPW,), jnp.int32),
        pltpu.VMEM((_BPW, _C), jnp.float32),
        pltpu.SemaphoreType.DMA,
    ],
    compiler_params=pltpu.CompilerParams(use_tc_tiling_on_sc=False,
                                         skip_device_barrier=True),
)
def _sc_gather(cb_hbm, idx_hbm, out_hbm, idx_v, rows_v, sem):
    wid = lax.axis_index("s") * _NC + lax.axis_index("c")
    base = wid * _BPW
    pltpu.sync_copy(idx_hbm.at[pl.ds(base, _BPW)], idx_v)
    pltpu.async_copy(cb_hbm.at[idx_v], rows_v, sem).wait()
    pltpu.sync_copy(rows_v, out_hbm.at[pl.ds(base, _BPW)])


@jax.jit
def kernel(x, codebook):
    xr = x.reshape(_N, _C, _P)
    idx_flat, loss = pl.pallas_call(
        _dist_body,
        out_shape=[
            jax.ShapeDtypeStruct((_B,), jnp.int32),
            jax.ShapeDtypeStruct((1, 1), jnp.float32),
        ],
    )(xr, codebook)
    rows = _sc_gather(codebook, idx_flat)
    loss = loss.reshape(())
    indices = idx_flat.reshape(_N, 16, 16)
    output = rows.reshape(_N, _P, _C).transpose(0, 2, 1).reshape(x.shape)
    return (loss, loss, indices, output)
